# baseline (speedup 1.0000x reference)
import functools

import jax
import jax.numpy as jnp
from jax import lax
from jax.experimental import pallas as pl
from jax.experimental.pallas import tpu as pltpu
from jax.experimental.pallas import tpu_sc as plsc

VOCAB = 1000000
EMBED_DIM = 64
BATCH = 16384
HIST = 50

_INFO = plsc.get_sparse_core_info()
NC = _INFO.num_cores
NS = _INFO.num_subcores
NW = NC * NS

A_COLS = 512
A_GRID = (VOCAB + A_COLS - 1) // A_COLS
A_HALF = 977
SPLIT = A_HALF * A_COLS

PITCH = 129


def _transpose_body(left_ref, right_ref, out_ref):
    out_ref[...] = jnp.concatenate(
        [left_ref[...].T, right_ref[...].T], axis=1
    )


def _pack_table(table_t):
    return pl.pallas_call(
        _transpose_body,
        grid=(A_HALF,),
        in_specs=[
            pl.BlockSpec((EMBED_DIM, A_COLS), lambda i: (0, i)),
            pl.BlockSpec((EMBED_DIM, A_COLS), lambda i: (0, i + A_HALF)),
        ],
        out_specs=pl.BlockSpec((A_COLS, 2 * EMBED_DIM), lambda i: (i, 0)),
        out_shape=jax.ShapeDtypeStruct((SPLIT, 2 * EMBED_DIM), jnp.float32),
    )(table_t, table_t)

CB = 128
TASKS = HIST * (BATCH // CB)
TPW = TASKS // NW
CPH = BATCH // CB

_mesh = plsc.VectorSubcoreMesh(core_axis_name="c", subcore_axis_name="s")


@functools.partial(
    pl.kernel,
    mesh=_mesh,
    out_type=jax.ShapeDtypeStruct((HIST, EMBED_DIM, BATCH), jnp.float32),
    compiler_params=pltpu.CompilerParams(needs_layout_passes=False),
    scratch_types=[
        pltpu.VMEM((2, CB), jnp.int32),
        pltpu.VMEM((2, CB), jnp.int32),
        pltpu.VMEM((2, CB), jnp.int32),
        pltpu.VMEM((2, CB, 2 * EMBED_DIM), jnp.float32),
        pltpu.VMEM((CB * PITCH,), jnp.float32),
        pltpu.VMEM((2, EMBED_DIM, CB), jnp.float32),
        pltpu.SemaphoreType.DMA,
        pltpu.SemaphoreType.DMA,
        pltpu.SemaphoreType.DMA,
    ],
)
def _gather_kernel(idx_hbm, t2_hbm, out_hbm, idx_v, row_v, off_v, rows_v,
                   stage_v, blk_v, gsem, osem, isem):
    wid = lax.axis_index("s") * NC + lax.axis_index("c")
    t0 = wid * TPW

    def fire(t, b):
        h = lax.div(t, CPH)
        c = lax.rem(t, CPH)
        pltpu.async_copy(
            idx_hbm.at[h, pl.ds(c * CB, CB)], idx_v.at[b], isem
        ).wait()
        for k in range(CB // 16):
            v = idx_v[b, pl.ds(16 * k, 16)]
            m = v >= SPLIT
            row_v[b, pl.ds(16 * k, 16)] = jnp.where(m, v - SPLIT, v)
            off_v[b, pl.ds(16 * k, 16)] = jnp.where(m, EMBED_DIM, 0)
        pltpu.async_copy(t2_hbm.at[row_v.at[b]], rows_v.at[b], gsem)

    def process(t, b):
        pltpu.make_async_copy(
            t2_hbm.at[row_v.at[b]], rows_v.at[b], gsem
        ).wait()
        def repitch(j, carry):
            for k in range(2 * EMBED_DIM // 16):
                stage_v[pl.ds(j * PITCH + 16 * k, 16)] = (
                    rows_v[b, j, pl.ds(16 * k, 16)]
                )
            return carry

        lax.fori_loop(0, CB, repitch, 0)
        iota = lax.iota(jnp.int32, 16)

        def col(d, carry):
            for g in range(CB // 16):
                base = (iota + g * 16) * PITCH + off_v[b, pl.ds(g * 16, 16)]
                blk_v[b, d, pl.ds(g * 16, 16)] = plsc.load_gather(
                    stage_v, [base + d]
                )
            return carry

        lax.fori_loop(0, EMBED_DIM, col, 0)
        h = lax.div(t, CPH)
        c = lax.rem(t, CPH)
        pltpu.async_copy(
            blk_v.at[b], out_hbm.at[h, :, pl.ds(c * CB, CB)], osem
        )

    def wait_store(b):
        pltpu.make_async_copy(
            blk_v.at[b], out_hbm.at[0, :, pl.ds(0, CB)], osem
        ).wait()

    fire(t0, 0)

    def body(i, carry):
        b = lax.rem(i, 2)
        nb = 1 - b

        @pl.when(i + 1 < TPW)
        def _():
            fire(t0 + i + 1, nb)

        process(t0 + i, b)

        @pl.when(i >= 1)
        def _():
            wait_store(nb)
        return carry

    lax.fori_loop(0, TPW, body, 0)
    wait_store((TPW - 1) % 2)


def kernel(dist, dist_embedder_weight):
    t2 = _pack_table(dist_embedder_weight.T)
    out_t = _gather_kernel(dist.T.astype(jnp.int32), t2)
    return out_t.transpose(2, 0, 1)


# speedup vs baseline: 2.1486x; 2.1486x over previous
import functools

import jax
import jax.numpy as jnp
from jax import lax
from jax.experimental import pallas as pl
from jax.experimental.pallas import tpu as pltpu
from jax.experimental.pallas import tpu_sc as plsc

VOCAB = 1000000
EMBED_DIM = 64
BATCH = 16384
HIST = 50

_INFO = plsc.get_sparse_core_info()
NC = _INFO.num_cores
NS = _INFO.num_subcores
NW = NC * NS

A_COLS = 512
A_HALF = 977
SPLIT = A_HALF * A_COLS

PITCH = 129


def _transpose_body(left_ref, right_ref, out_ref):
    out_ref[...] = jnp.concatenate(
        [left_ref[...].T, right_ref[...].T], axis=1
    )


def _pack_table(table_t):
    return pl.pallas_call(
        _transpose_body,
        grid=(A_HALF,),
        in_specs=[
            pl.BlockSpec((EMBED_DIM, A_COLS), lambda i: (0, i)),
            pl.BlockSpec((EMBED_DIM, A_COLS), lambda i: (0, i + A_HALF)),
        ],
        out_specs=pl.BlockSpec((A_COLS, 2 * EMBED_DIM), lambda i: (i, 0)),
        out_shape=jax.ShapeDtypeStruct((SPLIT, 2 * EMBED_DIM), jnp.float32),
    )(table_t, table_t)

CB = 128
TASKS = HIST * (BATCH // CB)
TPW = TASKS // NW
CPH = BATCH // CB

_mesh = plsc.VectorSubcoreMesh(core_axis_name="c", subcore_axis_name="s")


@functools.partial(
    pl.kernel,
    mesh=_mesh,
    out_type=jax.ShapeDtypeStruct((HIST, EMBED_DIM, BATCH), jnp.float32),
    compiler_params=pltpu.CompilerParams(needs_layout_passes=False),
    scratch_types=[
        pltpu.VMEM((2, CB), jnp.int32),
        pltpu.VMEM((2, CB), jnp.int32),
        pltpu.VMEM((2, CB), jnp.int32),
        pltpu.VMEM((2, CB, 2 * EMBED_DIM), jnp.float32),
        pltpu.VMEM((2, EMBED_DIM, CB), jnp.float32),
        pltpu.SemaphoreType.DMA,
        pltpu.SemaphoreType.DMA,
        pltpu.SemaphoreType.DMA,
    ],
)
def _gather_kernel(idx_hbm, t2_hbm, out_hbm, idx_v, row_v, off_v, rows_v,
                   blk_v, gsem, osem, isem):
    wid = lax.axis_index("s") * NC + lax.axis_index("c")
    t0 = wid * TPW

    def fire(t, b):
        h = lax.div(t, CPH)
        c = lax.rem(t, CPH)
        pltpu.async_copy(
            idx_hbm.at[h, pl.ds(c * CB, CB)], idx_v.at[b], isem
        ).wait()
        for k in range(CB // 16):
            v = idx_v[b, pl.ds(16 * k, 16)]
            m = v >= SPLIT
            row_v[b, pl.ds(16 * k, 16)] = jnp.where(m, v - SPLIT, v)
            off_v[b, pl.ds(16 * k, 16)] = jnp.where(m, EMBED_DIM, 0)
        pltpu.async_copy(t2_hbm.at[row_v.at[b]], rows_v.at[b], gsem)

    def process(t, b):
        pltpu.make_async_copy(
            t2_hbm.at[row_v.at[b]], rows_v.at[b], gsem
        ).wait()
        iota = lax.iota(jnp.int32, 16)
        dvecs = [jnp.bitwise_and(iota + k, 15) for k in range(16)]

        def grp(g, carry):
            rows16 = iota + g * 16
            off16 = off_v[b, pl.ds(g * 16, 16)]
            for q in range(EMBED_DIM // 16):
                base16 = off16 + 16 * q
                for k in range(16):
                    vals = plsc.load_gather(
                        rows_v.at[b], [rows16, base16 + dvecs[k]]
                    )
                    plsc.store_scatter(
                        blk_v.at[b], [dvecs[k] + 16 * q, rows16], vals
                    )
            return carry

        lax.fori_loop(0, CB // 16, grp, 0)
        h = lax.div(t, CPH)
        c = lax.rem(t, CPH)
        pltpu.async_copy(
            blk_v.at[b], out_hbm.at[h, :, pl.ds(c * CB, CB)], osem
        )

    def wait_store(b):
        pltpu.make_async_copy(
            blk_v.at[b], out_hbm.at[0, :, pl.ds(0, CB)], osem
        ).wait()

    fire(t0, 0)

    def body(i, carry):
        b = lax.rem(i, 2)
        nb = 1 - b

        @pl.when(i + 1 < TPW)
        def _():
            fire(t0 + i + 1, nb)

        process(t0 + i, b)

        @pl.when(i >= 1)
        def _():
            wait_store(nb)
        return carry

    lax.fori_loop(0, TPW, body, 0)
    wait_store((TPW - 1) % 2)


def kernel(dist, dist_embedder_weight):
    t2 = _pack_table(dist_embedder_weight.T)
    out_t = _gather_kernel(dist.T.astype(jnp.int32), t2)
    return out_t.transpose(2, 0, 1)


# speedup vs baseline: 2.1518x; 1.0015x over previous
import functools

import jax
import jax.numpy as jnp
from jax import lax
from jax.experimental import pallas as pl
from jax.experimental.pallas import tpu as pltpu
from jax.experimental.pallas import tpu_sc as plsc

VOCAB = 1000000
EMBED_DIM = 64
BATCH = 16384
HIST = 50

_INFO = plsc.get_sparse_core_info()
NC = _INFO.num_cores
NS = _INFO.num_subcores
NW = NC * NS

W = 128
SPLIT = 499968
N_PAIRS = SPLIT // W
ORPH = 999936
ORPH_T2 = 500032
T2_ROWS = ORPH_T2 + EMBED_DIM

_mesh_a = plsc.VectorSubcoreMesh(core_axis_name="c", subcore_axis_name="s")


@functools.partial(
    pl.kernel,
    mesh=_mesh_a,
    out_type=jax.ShapeDtypeStruct((T2_ROWS, 2 * EMBED_DIM), jnp.float32),
    compiler_params=pltpu.CompilerParams(needs_layout_passes=False),
    scratch_types=[
        pltpu.VMEM((2, EMBED_DIM, W), jnp.float32),
        pltpu.VMEM((2, EMBED_DIM, W), jnp.float32),
        pltpu.VMEM((2, W, 2 * EMBED_DIM), jnp.float32),
        pltpu.VMEM((EMBED_DIM, EMBED_DIM), jnp.float32),
        pltpu.SemaphoreType.DMA,
        pltpu.SemaphoreType.DMA,
    ],
)
def _pack_kernel(tt_hbm, orph_hbm, t2_hbm, lv, rv, tr_v, orph_v, gsem, osem):
    wid = lax.axis_index("s") * NC + lax.axis_index("c")
    iota = lax.iota(jnp.int32, 16)
    dvecs = [jnp.bitwise_and(iota + k, 15) for k in range(16)]
    npt = (N_PAIRS - wid + NW - 1) // NW

    def fire(s, b):
        lc = pl.multiple_of(s * W, 128)
        rc = pl.multiple_of(SPLIT + s * W, 128)
        pltpu.async_copy(tt_hbm.at[:, pl.ds(lc, W)], lv.at[b], gsem)
        pltpu.async_copy(tt_hbm.at[:, pl.ds(rc, W)], rv.at[b], gsem)

    def transpose_into(src, b, n16, colbase):
        def grp(g, carry):
            cols16 = iota + g * 16
            for q in range(EMBED_DIM // 16):
                for k in range(16):
                    vals = plsc.load_gather(
                        src.at[b], [dvecs[k] + 16 * q, cols16]
                    )
                    plsc.store_scatter(
                        tr_v.at[b],
                        [cols16, dvecs[k] + 16 * q + colbase],
                        vals,
                    )
            return carry

        lax.fori_loop(0, n16, grp, 0)

    def process(s, b):
        lc = pl.multiple_of(s * W, 128)
        rc = pl.multiple_of(SPLIT + s * W, 128)
        pltpu.make_async_copy(
            tt_hbm.at[:, pl.ds(lc, W)], lv.at[b], gsem
        ).wait()
        pltpu.make_async_copy(
            tt_hbm.at[:, pl.ds(rc, W)], rv.at[b], gsem
        ).wait()
        transpose_into(lv, b, W // 16, 0)
        transpose_into(rv, b, W // 16, EMBED_DIM)
        pltpu.async_copy(tr_v.at[b], t2_hbm.at[pl.ds(s * W, W)], osem)

    def wait_store(b):
        pltpu.make_async_copy(
            tr_v.at[b], t2_hbm.at[pl.ds(0, W)], osem
        ).wait()

    fire(wid, 0)

    def body(i, carry):
        b = lax.rem(i, 2)
        nb = 1 - b
        s = wid + i * NW

        @pl.when(i + 1 < npt)
        def _():
            fire(s + NW, nb)

        process(s, b)

        @pl.when(i >= 1)
        def _():
            wait_store(nb)
        return carry

    lax.fori_loop(0, npt, body, 0)
    wait_store((npt - 1) % 2)

    @pl.when(wid == 0)
    def _():
        pltpu.sync_copy(orph_hbm, orph_v)

        def cp(j, carry):
            for k in range(EMBED_DIM // 16):
                tr_v[0, j, pl.ds(16 * k, 16)] = orph_v[j, pl.ds(16 * k, 16)]
            return carry

        lax.fori_loop(0, EMBED_DIM, cp, 0)
        pltpu.sync_copy(
            tr_v.at[0, pl.ds(0, EMBED_DIM)],
            t2_hbm.at[pl.ds(ORPH_T2, EMBED_DIM)],
        )


def _pack_table(table_t, orph):
    return _pack_kernel(table_t, orph)

CB = 128
TASKS = HIST * (BATCH // CB)
TPW = TASKS // NW
CPH = BATCH // CB

_mesh = plsc.VectorSubcoreMesh(core_axis_name="c", subcore_axis_name="s")


@functools.partial(
    pl.kernel,
    mesh=_mesh,
    out_type=jax.ShapeDtypeStruct((HIST, EMBED_DIM, BATCH), jnp.float32),
    compiler_params=pltpu.CompilerParams(needs_layout_passes=False),
    scratch_types=[
        pltpu.VMEM((2, CB), jnp.int32),
        pltpu.VMEM((2, CB), jnp.int32),
        pltpu.VMEM((2, CB), jnp.int32),
        pltpu.VMEM((2, CB, 2 * EMBED_DIM), jnp.float32),
        pltpu.VMEM((2, EMBED_DIM, CB), jnp.float32),
        pltpu.SemaphoreType.DMA,
        pltpu.SemaphoreType.DMA,
        pltpu.SemaphoreType.DMA,
    ],
)
def _gather_kernel(idx_hbm, t2_hbm, out_hbm, idx_v, row_v, off_v, rows_v,
                   blk_v, gsem, osem, isem):
    wid = lax.axis_index("s") * NC + lax.axis_index("c")
    t0 = wid * TPW

    def fire(t, b):
        h = lax.div(t, CPH)
        c = lax.rem(t, CPH)
        pltpu.async_copy(
            idx_hbm.at[h, pl.ds(c * CB, CB)], idx_v.at[b], isem
        ).wait()
        for k in range(CB // 16):
            v = idx_v[b, pl.ds(16 * k, 16)]
            m = v >= SPLIT
            m2 = v >= ORPH
            row_v[b, pl.ds(16 * k, 16)] = jnp.where(
                m2, v - (ORPH - ORPH_T2), jnp.where(m, v - SPLIT, v)
            )
            off_v[b, pl.ds(16 * k, 16)] = jnp.where(
                m2, 0, jnp.where(m, EMBED_DIM, 0)
            )
        pltpu.async_copy(t2_hbm.at[row_v.at[b]], rows_v.at[b], gsem)

    def process(t, b):
        pltpu.make_async_copy(
            t2_hbm.at[row_v.at[b]], rows_v.at[b], gsem
        ).wait()
        iota = lax.iota(jnp.int32, 16)
        dvecs = [jnp.bitwise_and(iota + k, 15) for k in range(16)]

        def grp(g, carry):
            rows16 = iota + g * 16
            off16 = off_v[b, pl.ds(g * 16, 16)]
            for q in range(EMBED_DIM // 16):
                base16 = off16 + 16 * q
                for k in range(16):
                    vals = plsc.load_gather(
                        rows_v.at[b], [rows16, base16 + dvecs[k]]
                    )
                    plsc.store_scatter(
                        blk_v.at[b], [dvecs[k] + 16 * q, rows16], vals
                    )
            return carry

        lax.fori_loop(0, CB // 16, grp, 0)
        h = lax.div(t, CPH)
        c = lax.rem(t, CPH)
        pltpu.async_copy(
            blk_v.at[b], out_hbm.at[h, :, pl.ds(c * CB, CB)], osem
        )

    def wait_store(b):
        pltpu.make_async_copy(
            blk_v.at[b], out_hbm.at[0, :, pl.ds(0, CB)], osem
        ).wait()

    fire(t0, 0)

    def body(i, carry):
        b = lax.rem(i, 2)
        nb = 1 - b

        @pl.when(i + 1 < TPW)
        def _():
            fire(t0 + i + 1, nb)

        process(t0 + i, b)

        @pl.when(i >= 1)
        def _():
            wait_store(nb)
        return carry

    lax.fori_loop(0, TPW, body, 0)
    wait_store((TPW - 1) % 2)


def kernel(dist, dist_embedder_weight):
    t2 = _pack_table(
        dist_embedder_weight.T, dist_embedder_weight[ORPH:, :]
    )
    out_t = _gather_kernel(dist.T.astype(jnp.int32), t2)
    return out_t.transpose(2, 0, 1)
